# Initial kernel scaffold; baseline (speedup 1.0000x reference)
import jax
import jax.numpy as jnp
from jax.experimental import pallas as pl


def kernel(x, segment_ids):
    raise NotImplementedError("write your pallas kernel here")


# speedup vs baseline: 6.4566x; 6.4566x over previous
import functools

import jax
import jax.numpy as jnp
from jax import lax
from jax.experimental import pallas as pl
from jax.experimental.pallas import tpu as pltpu
from jax.experimental.pallas import tpu_sc as plsc

N = 320000
D = 128
S = 16
LANES = 16
NC = 2
NS = 16
NW = NC * NS
ROWS = N // NW
CH = 200
NCHUNK = ROWS // CH
NV = D // LANES
BSTEPS = 14


def _sc_partials(x, ids):
  mesh = plsc.VectorSubcoreMesh(core_axis_name="c", subcore_axis_name="s")

  @functools.partial(
      pl.kernel,
      out_type=jax.ShapeDtypeStruct((NW, S, D), jnp.float32),
      mesh=mesh,
      scratch_types=[
          pltpu.VMEM((CH, D), jnp.float32),
          pltpu.VMEM((ROWS + LANES,), jnp.int32),
          pltpu.VMEM((S, D), jnp.float32),
      ],
  )
  def k(x_hbm, ids_hbm, out_hbm, xbuf, idbuf, acc):
    wid = lax.axis_index("s") * NC + lax.axis_index("c")
    base = wid * ROWS
    neg_inf = jnp.full((LANES,), -jnp.inf, dtype=jnp.float32)
    for s in range(S):
      for j in range(NV):
        acc[s, pl.ds(j * LANES, LANES)] = neg_inf
    pltpu.sync_copy(ids_hbm.at[pl.ds(base, ROWS)], idbuf.at[pl.ds(0, ROWS)])
    idbuf[pl.ds(ROWS, LANES)] = jnp.full((LANES,), S, dtype=jnp.int32)

    def lower_bound(s):
      def bs_body(_, lo_hi):
        lo, hi = lo_hi
        mid = jnp.clip((lo + hi) // 2, 0, ROWS - 1)
        first = idbuf[pl.ds(mid, LANES)][0]
        below = first < s
        return jnp.where(below, mid, lo), jnp.where(below, hi, mid)

      lo, _ = lax.fori_loop(0, BSTEPS, bs_body, (-1, ROWS), unroll=False)
      return lo + 1

    bounds = [lower_bound(jnp.int32(s)) for s in range(1, S)] + [
        jnp.int32(ROWS)
    ]

    def chunk_body(c, _):
      c0 = c * CH
      pltpu.sync_copy(x_hbm.at[pl.ds(base + c0, CH), :], xbuf)
      lo_run = jnp.int32(0)
      for s in range(S):
        hi_run = bounds[s]
        start = jnp.clip(lo_run - c0, 0, CH)
        stop = jnp.clip(hi_run - c0, 0, CH)

        def row_body(i, carry, _s=s):
          return tuple(
              jnp.maximum(carry[j], xbuf[i, pl.ds(j * LANES, LANES)])
              for j in range(NV)
          )

        red = lax.fori_loop(
            start, stop, row_body, (neg_inf,) * NV, unroll=False
        )
        for j in range(NV):
          sl = pl.ds(j * LANES, LANES)
          acc[s, sl] = jnp.maximum(acc[s, sl], red[j])
        lo_run = hi_run
      return 0

    lax.fori_loop(0, NCHUNK, chunk_body, 0, unroll=False)
    pltpu.sync_copy(acc, out_hbm.at[wid])

  return k(x, ids)


def _combine(partials):
  def k2(p_ref, o_ref):
    o_ref[...] = jnp.max(p_ref[...], axis=0)

  return pl.pallas_call(
      k2,
      out_shape=jax.ShapeDtypeStruct((S, D), jnp.float32),
  )(partials)


@jax.jit
def kernel(x, segment_ids):
  ids = segment_ids.astype(jnp.int32)
  partials = _sc_partials(x, ids)
  return _combine(partials)
